# baseline (speedup 1.0000x reference)
import jax
import jax.numpy as jnp
from jax.experimental import pallas as pl

NUM_EMBEDDINGS = 1024
EMBEDDING_DIM = 64
COMMITMENT_COST = 0.25

_T = 1024
_N_TOKENS = 32 * 1024
_GRID = _N_TOKENS // _T
_INV_N = 1.0 / (_N_TOKENS * EMBEDDING_DIM)


def _vq_kernel(z_ref, e_ref, qst_ref, idx_ref, loss_ref):
    i = pl.program_id(0)
    z = z_ref[...]
    e = e_ref[...]
    zz = jnp.sum(z * z, axis=1, keepdims=True)
    e2 = jnp.sum(e * e, axis=1)
    mm = jax.lax.dot_general(
        z, e, dimension_numbers=(((1,), (1,)), ((), ())),
        preferred_element_type=jnp.float32)
    d = zz - 2.0 * mm + e2[None, :]
    dmin = jnp.min(d, axis=1, keepdims=True)
    iota = jax.lax.broadcasted_iota(jnp.int32, d.shape, 1)
    idx = jnp.min(jnp.where(d == dmin, iota, NUM_EMBEDDINGS), axis=1)
    idx_ref[0, 0, :] = idx
    onehot = (iota == idx[:, None]).astype(jnp.float32)
    q = jax.lax.dot_general(
        onehot, e, dimension_numbers=(((1,), (0,)), ((), ())),
        preferred_element_type=jnp.float32,
        precision=jax.lax.Precision.HIGHEST)
    qst_ref[...] = z + (q - z)
    part = jnp.sum(dmin, keepdims=True)

    @pl.when(i == 0)
    def _():
        loss_ref[...] = part

    @pl.when(i > 0)
    def _():
        loss_ref[...] += part

    @pl.when(i == _GRID - 1)
    def _():
        m = loss_ref[...] * _INV_N
        loss_ref[...] = m + COMMITMENT_COST * m


def kernel(inputs, embed_weight):
    flat = inputs.reshape(-1, EMBEDDING_DIM)
    qst, idx3, loss2 = pl.pallas_call(
        _vq_kernel,
        grid=(_GRID,),
        in_specs=[
            pl.BlockSpec((_T, EMBEDDING_DIM), lambda i: (i, 0)),
            pl.BlockSpec((NUM_EMBEDDINGS, EMBEDDING_DIM), lambda i: (0, 0)),
        ],
        out_specs=[
            pl.BlockSpec((_T, EMBEDDING_DIM), lambda i: (i, 0)),
            pl.BlockSpec((1, 1, _T), lambda i: (i, 0, 0)),
            pl.BlockSpec((1, 1), lambda i: (0, 0)),
        ],
        out_shape=[
            jax.ShapeDtypeStruct((_N_TOKENS, EMBEDDING_DIM), jnp.float32),
            jax.ShapeDtypeStruct((_GRID, 1, _T), jnp.int32),
            jax.ShapeDtypeStruct((1, 1), jnp.float32),
        ],
    )(flat, embed_weight)
    return (qst.reshape(inputs.shape), loss2[0, 0], idx3.reshape(-1))


# speedup vs baseline: 1.3884x; 1.3884x over previous
import functools

import jax
import jax.numpy as jnp
from jax import lax
from jax.experimental import pallas as pl
from jax.experimental.pallas import tpu as pltpu
from jax.experimental.pallas import tpu_sc as plsc

NUM_EMBEDDINGS = 1024
EMBEDDING_DIM = 64
COMMITMENT_COST = 0.25

_T = 1024
_N_TOKENS = 32 * 1024
_GRID = _N_TOKENS // _T
_INV_N = 1.0 / (_N_TOKENS * EMBEDDING_DIM)

_NC = 2
_NS = 16
_NW = _NC * _NS
_BPW = _N_TOKENS // _NW
_CH = 128
_PIECES = _BPW // _CH
_DPAD = 128


def _vq_tc_kernel(z_ref, e_ref, idx_ref, loss_ref):
    i = pl.program_id(0)
    z = z_ref[...]
    e = e_ref[...]
    zz = jnp.sum(z * z, axis=1, keepdims=True)
    e2 = jnp.sum(e * e, axis=1)
    mm = jax.lax.dot_general(
        z, e, dimension_numbers=(((1,), (1,)), ((), ())),
        preferred_element_type=jnp.float32)
    d = zz - 2.0 * mm + e2[None, :]
    dmin = jnp.min(d, axis=1, keepdims=True)
    iota = jax.lax.broadcasted_iota(jnp.int32, d.shape, 1)
    idx = jnp.min(jnp.where(d == dmin, iota, NUM_EMBEDDINGS), axis=1)
    idx_ref[0, 0, :] = idx
    part = jnp.sum(dmin, keepdims=True)

    @pl.when(i == 0)
    def _():
        loss_ref[...] = part

    @pl.when(i > 0)
    def _():
        loss_ref[...] += part

    @pl.when(i == _GRID - 1)
    def _():
        m = loss_ref[...] * _INV_N
        loss_ref[...] = m + COMMITMENT_COST * m


def _sc_gather_st(e_hbm, idx_hbm, z_hbm, out_hbm, idx_v, rows_v, z_v, sem):
    wid = lax.axis_index("s") * _NC + lax.axis_index("c")
    for p in range(_PIECES):
        base = wid * _BPW + p * _CH
        pltpu.sync_copy(idx_hbm.at[pl.ds(base, _CH)], idx_v)
        gather = pltpu.async_copy(e_hbm.at[idx_v], rows_v, sem)
        pltpu.sync_copy(z_hbm.at[pl.ds(base, _CH)], z_v)
        gather.wait()

        def body(t, carry):
            for j in range(EMBEDDING_DIM // 16):
                sl = pl.ds(j * 16, 16)
                q = rows_v[t, sl]
                zv = z_v[t, sl]
                z_v[t, sl] = zv + (q - zv)
            return carry

        lax.fori_loop(0, _CH, body, 0)
        pltpu.sync_copy(z_v, out_hbm.at[pl.ds(base, _CH)])


_sc_call = pl.kernel(
    _sc_gather_st,
    out_type=jax.ShapeDtypeStruct((_N_TOKENS, EMBEDDING_DIM), jnp.float32),
    mesh=plsc.VectorSubcoreMesh(core_axis_name="c", subcore_axis_name="s"),
    scratch_types=[
        pltpu.VMEM((_CH,), jnp.int32),
        pltpu.VMEM((_CH, _DPAD), jnp.float32),
        pltpu.VMEM((_CH, EMBEDDING_DIM), jnp.float32),
        pltpu.SemaphoreType.DMA,
    ],
)


def kernel(inputs, embed_weight):
    flat = inputs.reshape(-1, EMBEDDING_DIM)
    idx3, loss2 = pl.pallas_call(
        _vq_tc_kernel,
        grid=(_GRID,),
        in_specs=[
            pl.BlockSpec((_T, EMBEDDING_DIM), lambda i: (i, 0)),
            pl.BlockSpec((NUM_EMBEDDINGS, EMBEDDING_DIM), lambda i: (0, 0)),
        ],
        out_specs=[
            pl.BlockSpec((1, 1, _T), lambda i: (i, 0, 0)),
            pl.BlockSpec((1, 1), lambda i: (0, 0)),
        ],
        out_shape=[
            jax.ShapeDtypeStruct((_GRID, 1, _T), jnp.int32),
            jax.ShapeDtypeStruct((1, 1), jnp.float32),
        ],
    )(flat, embed_weight)
    idx = idx3.reshape(-1)
    e_pad = jnp.pad(embed_weight, ((0, 0), (0, _DPAD - EMBEDDING_DIM)))
    qst = _sc_call(e_pad, idx, flat)
    return (qst.reshape(inputs.shape), loss2[0, 0], idx)


# speedup vs baseline: 1.3909x; 1.0018x over previous
import functools

import jax
import jax.numpy as jnp
from jax import lax
from jax.experimental import pallas as pl
from jax.experimental.pallas import tpu as pltpu
from jax.experimental.pallas import tpu_sc as plsc

NUM_EMBEDDINGS = 1024
EMBEDDING_DIM = 64
COMMITMENT_COST = 0.25

_T = 1024
_N_TOKENS = 32 * 1024
_GRID = _N_TOKENS // _T
_INV_N = 1.0 / (_N_TOKENS * EMBEDDING_DIM)

_NC = 2
_NS = 16
_NW = _NC * _NS
_BPW = _N_TOKENS // _NW
_CH = 128
_PIECES = _BPW // _CH
_DPAD = 128


def _vq_tc_kernel(z_ref, e_ref, idx_ref, loss_ref, epad_ref):
    i = pl.program_id(0)
    z = z_ref[...]
    e = e_ref[...]
    zz = jnp.sum(z * z, axis=1, keepdims=True)
    e2 = jnp.sum(e * e, axis=1)
    mm = jax.lax.dot_general(
        z, e, dimension_numbers=(((1,), (1,)), ((), ())),
        preferred_element_type=jnp.float32)
    d = zz - 2.0 * mm + e2[None, :]
    dmin = jnp.min(d, axis=1, keepdims=True)
    iota = jax.lax.broadcasted_iota(jnp.int32, d.shape, 1)
    idx = jnp.min(jnp.where(d == dmin, iota, NUM_EMBEDDINGS), axis=1)
    idx_ref[...] = idx
    part = jnp.sum(dmin, keepdims=True)

    @pl.when(i == 0)
    def _():
        loss_ref[...] = part
        epad_ref[...] = jnp.concatenate(
            [e, jnp.zeros((NUM_EMBEDDINGS, _DPAD - EMBEDDING_DIM),
                          jnp.float32)], axis=1)

    @pl.when(i > 0)
    def _():
        loss_ref[...] += part

    @pl.when(i == _GRID - 1)
    def _():
        m = loss_ref[...] * _INV_N
        loss_ref[...] = m + COMMITMENT_COST * m


def _sc_gather_st(e_hbm, idx_hbm, z_hbm, out_hbm, idx_v, rows_v, z_v, sem):
    wid = lax.axis_index("s") * _NC + lax.axis_index("c")
    for p in range(_PIECES):
        base = wid * _BPW + p * _CH
        pltpu.sync_copy(idx_hbm.at[pl.ds(base, _CH)], idx_v)
        gather = pltpu.async_copy(e_hbm.at[idx_v], rows_v, sem)
        pltpu.sync_copy(z_hbm.at[pl.ds(base, _CH)], z_v)
        gather.wait()

        def body(t, carry):
            for j in range(EMBEDDING_DIM // 16):
                sl = pl.ds(j * 16, 16)
                q = rows_v[t, sl]
                zv = z_v[t, sl]
                z_v[t, sl] = zv + (q - zv)
            return carry

        lax.fori_loop(0, _CH, body, 0)
        pltpu.sync_copy(z_v, out_hbm.at[pl.ds(base, _CH)])


_sc_call = pl.kernel(
    _sc_gather_st,
    out_type=jax.ShapeDtypeStruct((_N_TOKENS, EMBEDDING_DIM), jnp.float32),
    mesh=plsc.VectorSubcoreMesh(core_axis_name="c", subcore_axis_name="s"),
    scratch_types=[
        pltpu.VMEM((_CH,), jnp.int32),
        pltpu.VMEM((_CH, _DPAD), jnp.float32),
        pltpu.VMEM((_CH, EMBEDDING_DIM), jnp.float32),
        pltpu.SemaphoreType.DMA,
    ],
)


def kernel(inputs, embed_weight):
    flat = inputs.reshape(-1, EMBEDDING_DIM)
    idx, loss2, e_pad = pl.pallas_call(
        _vq_tc_kernel,
        grid=(_GRID,),
        in_specs=[
            pl.BlockSpec((_T, EMBEDDING_DIM), lambda i: (i, 0)),
            pl.BlockSpec((NUM_EMBEDDINGS, EMBEDDING_DIM), lambda i: (0, 0)),
        ],
        out_specs=[
            pl.BlockSpec((_T,), lambda i: (i,)),
            pl.BlockSpec((1, 1), lambda i: (0, 0)),
            pl.BlockSpec((NUM_EMBEDDINGS, _DPAD), lambda i: (0, 0)),
        ],
        out_shape=[
            jax.ShapeDtypeStruct((_N_TOKENS,), jnp.int32),
            jax.ShapeDtypeStruct((1, 1), jnp.float32),
            jax.ShapeDtypeStruct((NUM_EMBEDDINGS, _DPAD), jnp.float32),
        ],
    )(flat, embed_weight)
    qst = _sc_call(e_pad, idx, flat)
    return (qst.reshape(inputs.shape), loss2[0, 0], idx)
